# baseline (speedup 1.0000x reference)
import functools

import jax
import jax.numpy as jnp
from jax import lax
from jax.experimental import pallas as pl
from jax.experimental.pallas import tpu as pltpu
from jax.experimental.pallas import tpu_sc as plsc

_N_TOKENS = 8192
_N_EMBED = 2048
_N_EXPERTS = 16
_NW = 32
_TPW = _N_TOKENS // _NW
_L = 16
_GROUPS = _TPW // _L


def _tc_body(x_ref, w_ref, wn_ref, b_ref, bn_ref, eps_ref, out_ref):
    wcat = jnp.concatenate([w_ref[...], wn_ref[...]], axis=0)
    acc = lax.dot_general(
        wcat.astype(jnp.bfloat16), x_ref[...].astype(jnp.bfloat16),
        (((1,), (1,)), ((), ())),
        preferred_element_type=jnp.float32,
    )
    logits = acc[0:_N_EXPERTS, :] + b_ref[...].reshape(_N_EXPERTS, 1)
    noise = acc[_N_EXPERTS:, :] + bn_ref[...].reshape(_N_EXPERTS, 1)
    for c in range(_CHUNKS_PER_BLOCK):
        lo, hi = c * _TPW, (c + 1) * _TPW
        out_ref[c] = (logits[:, lo:hi]
                      + eps_ref[c] * jax.nn.softplus(noise[:, lo:hi]))


_CHUNKS_PER_BLOCK = 4


def _noisy_logits(x, w, wn, b, bn, epsb, nchunks, chunk_off):
    cpb = _CHUNKS_PER_BLOCK
    blk = _TPW * cpb
    boff = chunk_off // cpb
    return pl.pallas_call(
        _tc_body,
        grid=(nchunks // cpb,),
        in_specs=[
            pl.BlockSpec((blk, _N_EMBED), lambda w: (w + boff, 0)),
            pl.BlockSpec((_N_EXPERTS, _N_EMBED), lambda w: (0, 0)),
            pl.BlockSpec((_N_EXPERTS, _N_EMBED), lambda w: (0, 0)),
            pl.BlockSpec((1, _N_EXPERTS), lambda w: (0, 0)),
            pl.BlockSpec((1, _N_EXPERTS), lambda w: (0, 0)),
            pl.BlockSpec((cpb, _N_EXPERTS, _TPW), lambda w: (w, 0, 0)),
        ],
        out_specs=pl.BlockSpec((cpb, _N_EXPERTS, _TPW), lambda w: (w, 0, 0)),
        out_shape=jax.ShapeDtypeStruct((nchunks, _N_EXPERTS, _TPW), jnp.float32),
        compiler_params=pltpu.CompilerParams(
            dimension_semantics=("parallel",)),
    )(x, w, wn, b, bn, epsb)


@functools.cache
def _sc_router(phases):
    ntok = _N_TOKENS // phases
    tpv = ntok // _NW
    groups = tpv // _L
    parts = _TPW // tpv

    def body(noisy_hbm, probs_hbm, idx_hbm, nl_v, probs_v, idx_v):
        w = lax.axis_index("s") * 2 + lax.axis_index("c")
        base = w * tpv
        if parts == 1:
            pltpu.sync_copy(noisy_hbm.at[w], nl_v)
        else:
            chunk = w // parts
            part = w % parts
            pltpu.sync_copy(
                noisy_hbm.at[chunk, :, pl.ds(part * tpv, tpv)], nl_v)

        zeros_f = jnp.zeros((_L,), jnp.float32)

        def _zero_row(i, carry):
            probs_v[i, :] = zeros_f
            return carry

        lax.fori_loop(0, tpv, _zero_row, 0)

        lane = lax.iota(jnp.int32, _L)
        zeros_i = jnp.zeros((_L,), jnp.int32)
        ones_i = jnp.full((_L,), 1, jnp.int32)
        neg_inf = jnp.full((_L,), -jnp.inf, jnp.float32)

        def _group(g, carry):
            t_vec = g * _L + lane
            max1 = nl_v[0, pl.ds(g * _L, _L)]
            idx1 = zeros_i
            max2 = neg_inf
            idx2 = zeros_i
            for e in range(1, _N_EXPERTS):
                xe = nl_v[e, pl.ds(g * _L, _L)]
                evec = jnp.full((_L,), e, jnp.int32)
                gt1 = xe > max1
                gt2 = xe > max2
                max2 = jnp.where(gt1, max1, jnp.where(gt2, xe, max2))
                idx2 = jnp.where(gt1, idx1, jnp.where(gt2, evec, idx2))
                max1 = jnp.where(gt1, xe, max1)
                idx1 = jnp.where(gt1, evec, idx1)
            e2 = jnp.exp(max2 - max1)
            p1 = 1.0 / (1.0 + e2)
            p2 = e2 * p1
            plsc.store_scatter(probs_v, [t_vec, idx1], p1)
            plsc.store_scatter(probs_v, [t_vec, idx2], p2)
            plsc.store_scatter(idx_v, [t_vec, zeros_i], idx1)
            plsc.store_scatter(idx_v, [t_vec, ones_i], idx2)
            return carry

        lax.fori_loop(0, groups, _group, 0)

        pltpu.sync_copy(probs_v, probs_hbm.at[pl.ds(base, tpv)])
        pltpu.sync_copy(idx_v, idx_hbm.at[pl.ds(base, tpv)])

    return pl.kernel(
        body,
        out_type=(
            jax.ShapeDtypeStruct((ntok, _N_EXPERTS), jnp.float32),
            jax.ShapeDtypeStruct((ntok, 2), jnp.int32),
        ),
        mesh=plsc.VectorSubcoreMesh(core_axis_name="c", subcore_axis_name="s"),
        compiler_params=pltpu.CompilerParams(needs_layout_passes=False),
        scratch_types=[
            pltpu.VMEM((_N_EXPERTS, tpv), jnp.float32),
            pltpu.VMEM((tpv, _N_EXPERTS), jnp.float32),
            pltpu.VMEM((tpv, 2), jnp.int32),
        ],
    )


@functools.cache
def _eps_blocked():
    eps = jax.random.normal(
        jax.random.key(42), (_N_TOKENS, _N_EXPERTS), dtype=jnp.float32)
    return jax.device_get(
        eps.T.reshape(_N_EXPERTS, _NW, _TPW).transpose(1, 0, 2))


_PHASES = 2


def kernel(mh_output, W, b, W_noise, b_noise):
    epsb = _eps_blocked()
    b2, bn2 = b[None, :], b_noise[None, :]
    cpp = _NW // _PHASES
    sc = _sc_router(_PHASES)
    probs_parts, idx_parts = [], []
    for ph in range(_PHASES):
        eph = jnp.asarray(epsb[ph * cpp:(ph + 1) * cpp])
        noisy = _noisy_logits(mh_output, W, W_noise, b2, bn2, eph,
                              cpp, ph * cpp)
        p, i = sc(noisy)
        probs_parts.append(p)
        idx_parts.append(i)
    routing_probs = jnp.concatenate(probs_parts, axis=0)
    top_k_idx = jnp.concatenate(idx_parts, axis=0)
    return routing_probs, top_k_idx


# speedup vs baseline: 1.1031x; 1.1031x over previous
import functools

import jax
import jax.numpy as jnp
from jax import lax
from jax.experimental import pallas as pl
from jax.experimental.pallas import tpu as pltpu
from jax.experimental.pallas import tpu_sc as plsc

_N_TOKENS = 8192
_N_EMBED = 2048
_N_EXPERTS = 16
_NW = 32
_TPW = _N_TOKENS // _NW
_L = 16
_GROUPS = _TPW // _L


def _tc_body(x_ref, w_ref, wn_ref, b_ref, bn_ref, eps_ref, out_ref):
    wcat = jnp.concatenate([w_ref[...], wn_ref[...]], axis=0)
    acc = lax.dot_general(
        wcat.astype(jnp.bfloat16), x_ref[...].astype(jnp.bfloat16),
        (((1,), (1,)), ((), ())),
        preferred_element_type=jnp.float32,
    )
    logits = acc[0:_N_EXPERTS, :] + b_ref[...].reshape(_N_EXPERTS, 1)
    noise = acc[_N_EXPERTS:, :] + bn_ref[...].reshape(_N_EXPERTS, 1)
    for c in range(_CHUNKS_PER_BLOCK):
        lo, hi = c * _TPW, (c + 1) * _TPW
        out_ref[c] = (logits[:, lo:hi]
                      + eps_ref[c] * jax.nn.softplus(noise[:, lo:hi]))


_CHUNKS_PER_BLOCK = 4


def _noisy_logits(x, w, wn, b, bn, epsb, nchunks, chunk_off):
    cpb = _CHUNKS_PER_BLOCK
    blk = _TPW * cpb
    boff = chunk_off // cpb
    return pl.pallas_call(
        _tc_body,
        grid=(nchunks // cpb,),
        in_specs=[
            pl.BlockSpec((blk, _N_EMBED), lambda w: (w + boff, 0)),
            pl.BlockSpec((_N_EXPERTS, _N_EMBED), lambda w: (0, 0)),
            pl.BlockSpec((_N_EXPERTS, _N_EMBED), lambda w: (0, 0)),
            pl.BlockSpec((1, _N_EXPERTS), lambda w: (0, 0)),
            pl.BlockSpec((1, _N_EXPERTS), lambda w: (0, 0)),
            pl.BlockSpec((cpb, _N_EXPERTS, _TPW), lambda w: (w, 0, 0)),
        ],
        out_specs=pl.BlockSpec((cpb, _N_EXPERTS, _TPW), lambda w: (w, 0, 0)),
        out_shape=jax.ShapeDtypeStruct((nchunks, _N_EXPERTS, _TPW), jnp.float32),
        compiler_params=pltpu.CompilerParams(
            dimension_semantics=("parallel",)),
    )(x, w, wn, b, bn, epsb)


@functools.cache
def _sc_router(phases):
    ntok = _N_TOKENS // phases
    tpv = ntok // _NW
    groups = tpv // _L
    parts = _TPW // tpv

    def body(noisy_hbm, probs_hbm, idx_hbm, nl_v, probs_v, idx_v):
        w = lax.axis_index("s") * 2 + lax.axis_index("c")
        base = w * tpv
        if parts == 1:
            pltpu.sync_copy(noisy_hbm.at[w], nl_v)
        else:
            chunk = w // parts
            part = w % parts
            pltpu.sync_copy(
                noisy_hbm.at[chunk, :, pl.ds(part * tpv, tpv)], nl_v)

        zeros_f = jnp.zeros((_L,), jnp.float32)

        def _zero_row(i, carry):
            probs_v[i, :] = zeros_f
            return carry

        lax.fori_loop(0, tpv, _zero_row, 0)

        lane = lax.iota(jnp.int32, _L)
        zeros_i = jnp.zeros((_L,), jnp.int32)
        ones_i = jnp.full((_L,), 1, jnp.int32)
        neg_inf = jnp.full((_L,), -jnp.inf, jnp.float32)

        def _group(g, carry):
            t_vec = g * _L + lane
            max1 = nl_v[0, pl.ds(g * _L, _L)]
            idx1 = zeros_i
            max2 = neg_inf
            idx2 = zeros_i
            for e in range(1, _N_EXPERTS):
                xe = nl_v[e, pl.ds(g * _L, _L)]
                evec = jnp.full((_L,), e, jnp.int32)
                gt1 = xe > max1
                gt2 = xe > max2
                max2 = jnp.where(gt1, max1, jnp.where(gt2, xe, max2))
                idx2 = jnp.where(gt1, idx1, jnp.where(gt2, evec, idx2))
                max1 = jnp.where(gt1, xe, max1)
                idx1 = jnp.where(gt1, evec, idx1)
            e2 = jnp.exp(max2 - max1)
            p1 = 1.0 / (1.0 + e2)
            p2 = e2 * p1
            plsc.store_scatter(probs_v, [t_vec, idx1], p1)
            plsc.store_scatter(probs_v, [t_vec, idx2], p2)
            plsc.store_scatter(idx_v, [t_vec, zeros_i], idx1)
            plsc.store_scatter(idx_v, [t_vec, ones_i], idx2)
            return carry

        lax.fori_loop(0, groups, _group, 0)

        pltpu.sync_copy(probs_v, probs_hbm.at[pl.ds(base, tpv)])
        pltpu.sync_copy(idx_v, idx_hbm.at[pl.ds(base, tpv)])

    return pl.kernel(
        body,
        out_type=(
            jax.ShapeDtypeStruct((ntok, _N_EXPERTS), jnp.float32),
            jax.ShapeDtypeStruct((ntok, 2), jnp.int32),
        ),
        mesh=plsc.VectorSubcoreMesh(core_axis_name="c", subcore_axis_name="s"),
        compiler_params=pltpu.CompilerParams(needs_layout_passes=False),
        scratch_types=[
            pltpu.VMEM((_N_EXPERTS, tpv), jnp.float32),
            pltpu.VMEM((tpv, _N_EXPERTS), jnp.float32),
            pltpu.VMEM((tpv, 2), jnp.int32),
        ],
    )


@functools.cache
def _eps_blocked():
    with jax.ensure_compile_time_eval():
        eps = jax.random.normal(
            jax.random.key(42), (_N_TOKENS, _N_EXPERTS), dtype=jnp.float32)
        blocked = eps.T.reshape(_N_EXPERTS, _NW, _TPW).transpose(1, 0, 2)
    return jax.device_get(blocked)


def kernel(mh_output, W, b, W_noise, b_noise):
    epsb = jnp.asarray(_eps_blocked())
    noisy = _noisy_logits(mh_output, W, W_noise, b[None, :], b_noise[None, :],
                          epsb, _NW, 0)
    routing_probs, top_k_idx = _sc_router(1)(noisy)
    return routing_probs, top_k_idx


# speedup vs baseline: 1.3425x; 1.2170x over previous
import functools

import jax
import jax.numpy as jnp
from jax import lax
from jax.experimental import pallas as pl
from jax.experimental.pallas import tpu as pltpu
from jax.experimental.pallas import tpu_sc as plsc

_N_TOKENS = 8192
_N_EMBED = 2048
_N_EXPERTS = 16
_NW = 32
_TPW = _N_TOKENS // _NW
_L = 16
_GROUPS = _TPW // _L


def _tc_body(x_ref, w_ref, wn_ref, b_ref, bn_ref, eps_ref, out_ref):
    wcat = jnp.concatenate([w_ref[...], wn_ref[...]], axis=0)
    acc = lax.dot_general(
        wcat.astype(jnp.bfloat16), x_ref[...].astype(jnp.bfloat16),
        (((1,), (1,)), ((), ())),
        preferred_element_type=jnp.float32,
    )
    logits = acc[0:_N_EXPERTS, :] + b_ref[...].reshape(_N_EXPERTS, 1)
    noise = acc[_N_EXPERTS:, :] + bn_ref[...].reshape(_N_EXPERTS, 1)
    for c in range(_CHUNKS_PER_BLOCK):
        lo, hi = c * _TPW, (c + 1) * _TPW
        out_ref[c] = (logits[:, lo:hi]
                      + eps_ref[c] * jax.nn.softplus(noise[:, lo:hi]))


_CHUNKS_PER_BLOCK = 4


def _noisy_logits(x, w, wn, b, bn, epsb, nchunks, chunk_off):
    cpb = _CHUNKS_PER_BLOCK
    blk = _TPW * cpb
    boff = chunk_off // cpb
    return pl.pallas_call(
        _tc_body,
        grid=(nchunks // cpb,),
        in_specs=[
            pl.BlockSpec((blk, _N_EMBED), lambda w: (w + boff, 0)),
            pl.BlockSpec((_N_EXPERTS, _N_EMBED), lambda w: (0, 0)),
            pl.BlockSpec((_N_EXPERTS, _N_EMBED), lambda w: (0, 0)),
            pl.BlockSpec((1, _N_EXPERTS), lambda w: (0, 0)),
            pl.BlockSpec((1, _N_EXPERTS), lambda w: (0, 0)),
            pl.BlockSpec((cpb, _N_EXPERTS, _TPW), lambda w: (w, 0, 0)),
        ],
        out_specs=pl.BlockSpec((cpb, _N_EXPERTS, _TPW), lambda w: (w, 0, 0)),
        out_shape=jax.ShapeDtypeStruct((nchunks, _N_EXPERTS, _TPW), jnp.float32),
        compiler_params=pltpu.CompilerParams(
            dimension_semantics=("parallel",)),
    )(x, w, wn, b, bn, epsb)


@functools.cache
def _sc_router(phases):
    ntok = _N_TOKENS // phases
    tpv = ntok // _NW
    groups = tpv // _L
    parts = _TPW // tpv

    def body(noisy_hbm, probs_hbm, idx_hbm, nl_v, probs_v, idx_v):
        w = lax.axis_index("s") * 2 + lax.axis_index("c")
        base = w * tpv
        if parts == 1:
            pltpu.sync_copy(noisy_hbm.at[w], nl_v)
        else:
            chunk = w // parts
            part = w % parts
            pltpu.sync_copy(
                noisy_hbm.at[chunk, :, pl.ds(part * tpv, tpv)], nl_v)

        zeros_f = jnp.zeros((_L,), jnp.float32)

        def _zero_row(i, carry):
            for e in range(_N_EXPERTS):
                probs_v[e, pl.ds(i * _L, _L)] = zeros_f
            return carry

        lax.fori_loop(0, tpv // _L, _zero_row, 0)

        lane = lax.iota(jnp.int32, _L)
        zeros_i = jnp.zeros((_L,), jnp.int32)
        ones_i = jnp.full((_L,), 1, jnp.int32)
        neg_inf = jnp.full((_L,), -jnp.inf, jnp.float32)

        def _group(g, carry):
            t_vec = g * _L + lane
            max1 = nl_v[0, pl.ds(g * _L, _L)]
            idx1 = zeros_i
            max2 = neg_inf
            idx2 = zeros_i
            for e in range(1, _N_EXPERTS):
                xe = nl_v[e, pl.ds(g * _L, _L)]
                evec = jnp.full((_L,), e, jnp.int32)
                gt1 = xe > max1
                gt2 = xe > max2
                max2 = jnp.where(gt1, max1, jnp.where(gt2, xe, max2))
                idx2 = jnp.where(gt1, idx1, jnp.where(gt2, evec, idx2))
                max1 = jnp.where(gt1, xe, max1)
                idx1 = jnp.where(gt1, evec, idx1)
            e2 = jnp.exp(max2 - max1)
            p1 = 1.0 / (1.0 + e2)
            p2 = e2 * p1
            plsc.store_scatter(probs_v, [idx1, t_vec], p1)
            plsc.store_scatter(probs_v, [idx2, t_vec], p2)
            idx_v[0, pl.ds(g * _L, _L)] = idx1
            idx_v[1, pl.ds(g * _L, _L)] = idx2
            return carry

        lax.fori_loop(0, groups, _group, 0)

        pltpu.sync_copy(probs_v, probs_hbm.at[:, pl.ds(base, tpv)])
        pltpu.sync_copy(idx_v, idx_hbm.at[:, pl.ds(base, tpv)])

    return pl.kernel(
        body,
        out_type=(
            jax.ShapeDtypeStruct((_N_EXPERTS, ntok), jnp.float32),
            jax.ShapeDtypeStruct((2, ntok), jnp.int32),
        ),
        mesh=plsc.VectorSubcoreMesh(core_axis_name="c", subcore_axis_name="s"),
        compiler_params=pltpu.CompilerParams(needs_layout_passes=False),
        scratch_types=[
            pltpu.VMEM((_N_EXPERTS, tpv), jnp.float32),
            pltpu.VMEM((_N_EXPERTS, tpv), jnp.float32),
            pltpu.VMEM((2, tpv), jnp.int32),
        ],
    )


@functools.cache
def _eps_blocked():
    with jax.ensure_compile_time_eval():
        eps = jax.random.normal(
            jax.random.key(42), (_N_TOKENS, _N_EXPERTS), dtype=jnp.float32)
        blocked = eps.T.reshape(_N_EXPERTS, _NW, _TPW).transpose(1, 0, 2)
    return jax.device_get(blocked)


def kernel(mh_output, W, b, W_noise, b_noise):
    epsb = jnp.asarray(_eps_blocked())
    noisy = _noisy_logits(mh_output, W, W_noise, b[None, :], b_noise[None, :],
                          epsb, _NW, 0)
    probs_t, idx_t = _sc_router(1)(noisy)
    return probs_t.T, idx_t.T
